# baseline (speedup 1.0000x reference)
import functools

import jax
import jax.numpy as jnp
from jax import lax
from jax.experimental import pallas as pl
from jax.experimental.pallas import tpu as pltpu
from jax.experimental.pallas import tpu_sc as plsc

VOCAB1 = 100001
D = 64
B, L = 4096, 50
NC, NS = 2, 16
NW = NC * NS
K = 2
B_PER_C = B // K
B_PER_W = B_PER_C // NW
CHB = 16
NCH = B_PER_W // CHB


def _make_body(slab):
    def body(table_hbm, idx_hbm, out_hbm, idx_v, rows_v, g0, g1, w0, w1):
        _gather_grid(
            slab, table_hbm, idx_hbm, out_hbm, idx_v, rows_v, g0, g1, w0, w1
        )
    return body


def _gather_grid(slab, table_hbm, idx_hbm, out_hbm, idx_v, rows_v, g0, g1, w0, w1):
    wid = lax.axis_index("s") * NC + lax.axis_index("c")
    bbase = wid * B_PER_W
    ibase = slab * B_PER_C + bbase
    gsem = (g0, g1)
    wsem = (w0, w1)

    pltpu.sync_copy(idx_hbm.at[pl.ds(ibase, B_PER_W)], idx_v)

    def start_gathers(j, b):
        return [
            pltpu.async_copy(
                table_hbm.at[idx_v.at[j * CHB + k]],
                rows_v.at[b].at[k],
                gsem[b],
            )
            for k in range(CHB)
        ]

    def start_writeback(j, b):
        return pltpu.async_copy(
            rows_v.at[b],
            out_hbm.at[pl.ds(bbase + j * CHB, CHB)],
            wsem[b],
        )

    gh = [None] * NCH
    wh = [None] * NCH
    gh[0] = start_gathers(0, 0)
    for j in range(NCH):
        b = j % 2
        for h in gh[j]:
            h.wait()
        wh[j] = start_writeback(j, b)
        if j + 1 < NCH:
            if j >= 1:
                wh[j - 1].wait()
            gh[j + 1] = start_gathers(j + 1, 1 - b)
    wh[NCH - 2].wait()
    wh[NCH - 1].wait()


def kernel(indices, vectors):
    idx = indices.astype(jnp.int32)
    mesh = plsc.VectorSubcoreMesh(core_axis_name="c", subcore_axis_name="s")
    outs = []
    for slab in range(K):
        run = functools.partial(
            pl.kernel,
            mesh=mesh,
            compiler_params=pltpu.CompilerParams(use_tc_tiling_on_sc=False),
            out_type=jax.ShapeDtypeStruct((B_PER_C, L, D), jnp.float32),
            scratch_types=[
                pltpu.VMEM((B_PER_W, L), jnp.int32),
                pltpu.VMEM((2, CHB, L, D), jnp.float32),
                pltpu.SemaphoreType.DMA,
                pltpu.SemaphoreType.DMA,
                pltpu.SemaphoreType.DMA,
                pltpu.SemaphoreType.DMA,
            ],
        )(_make_body(slab))
        outs.append(run(vectors, idx))
    return jnp.concatenate(outs, axis=0)


# speedup vs baseline: 1.3328x; 1.3328x over previous
import functools

import jax
import jax.numpy as jnp
from jax import lax
from jax.experimental import pallas as pl
from jax.experimental.pallas import tpu as pltpu
from jax.experimental.pallas import tpu_sc as plsc

VOCAB1 = 100001
D = 64
DP = 128
B, L = 4096, 50
NC, NS = 2, 16
NW = NC * NS
B_PER_W = B // NW
CHB = 8
NCH = B_PER_W // CHB


def _gather_grid(table_hbm, idx_hbm, out_hbm, idx_v, rows_v, g0, g1, w0, w1):
    wid = lax.axis_index("s") * NC + lax.axis_index("c")
    bbase = wid * B_PER_W
    gsem = (g0, g1)
    wsem = (w0, w1)

    pltpu.sync_copy(idx_hbm.at[pl.ds(bbase, B_PER_W)], idx_v)

    def start_gathers(j, b):
        return [
            pltpu.async_copy(
                table_hbm.at[idx_v.at[j * CHB + k]],
                rows_v.at[b].at[k],
                gsem[b],
            )
            for k in range(CHB)
        ]

    def start_writeback(j, b):
        return pltpu.async_copy(
            rows_v.at[b],
            out_hbm.at[pl.ds(bbase + j * CHB, CHB)],
            wsem[b],
        )

    gh = [None] * NCH
    wh = [None] * NCH
    gh[0] = start_gathers(0, 0)
    for j in range(NCH):
        b = j % 2
        for h in gh[j]:
            h.wait()
        wh[j] = start_writeback(j, b)
        if j + 1 < NCH:
            if j >= 1:
                wh[j - 1].wait()
            gh[j + 1] = start_gathers(j + 1, 1 - b)
    wh[NCH - 2].wait()
    wh[NCH - 1].wait()


def kernel(indices, vectors):
    idx = indices.astype(jnp.int32)
    table = jnp.pad(vectors, ((0, 0), (0, DP - D)))
    mesh = plsc.VectorSubcoreMesh(core_axis_name="c", subcore_axis_name="s")
    run = functools.partial(
        pl.kernel,
        mesh=mesh,
        out_type=jax.ShapeDtypeStruct((B, L, DP), jnp.float32),
        scratch_types=[
            pltpu.VMEM((B_PER_W, L), jnp.int32),
            pltpu.VMEM((2, CHB, L, DP), jnp.float32),
            pltpu.SemaphoreType.DMA,
            pltpu.SemaphoreType.DMA,
            pltpu.SemaphoreType.DMA,
            pltpu.SemaphoreType.DMA,
        ],
    )(_gather_grid)
    return run(table, idx)[:, :, :D]
